# baseline (speedup 1.0000x reference)
import jax
import jax.numpy as jnp
from jax.experimental import pallas as pl
from jax.experimental.pallas import tpu as pltpu
from jax.experimental.pallas import tpu_sc as plsc

_K = 8192
_H = 256
_TM = 256
_TN = 1024
_BETA = 0.25
_GW = 128


def _argmin_body(z_ref, eT_ref, idx_ref, bd_ref):
    z = z_ref[...]
    z2 = jnp.sum(z * z, axis=1)

    def step(j, carry):
        best_d, best_i = carry
        eT = eT_ref[:, pl.ds(j * _TN, _TN)]
        prod = jnp.dot(z, eT, preferred_element_type=jnp.float32)
        e2 = jnp.sum(eT * eT, axis=0)
        dist = (z2[:, None] + e2[None, :]) - 2.0 * prod
        cmin = jnp.min(dist, axis=1)
        carg = jnp.argmin(dist, axis=1).astype(jnp.int32) + j * _TN
        better = cmin < best_d
        return (jnp.where(better, cmin, best_d),
                jnp.where(better, carg, best_i))

    init = (jnp.full((_TM,), jnp.inf, jnp.float32),
            jnp.zeros((_TM,), jnp.int32))
    best_d, best_i = jax.lax.fori_loop(0, _K // _TN, step, init)
    idx_ref[...] = best_i.reshape(1, 1, _TM)
    bd_ref[...] = best_d.reshape(1, 1, _TM)


def _argmin_call(zf, eT):
    n_tiles = zf.shape[0] // _TM
    return pl.pallas_call(
        _argmin_body,
        grid=(n_tiles,),
        in_specs=[
            pl.BlockSpec((_TM, _H), lambda i: (i, 0)),
            pl.BlockSpec((_H, _K), lambda i: (0, 0)),
        ],
        out_specs=[
            pl.BlockSpec((1, 1, _TM), lambda i: (i, 0, 0)),
            pl.BlockSpec((1, 1, _TM), lambda i: (i, 0, 0)),
        ],
        out_shape=[
            jax.ShapeDtypeStruct((n_tiles, 1, _TM), jnp.int32),
            jax.ShapeDtypeStruct((n_tiles, 1, _TM), jnp.float32),
        ],
    )(zf, eT)


def _sc_gather(emb, idx):
    n = idx.shape[0]
    mesh = plsc.VectorSubcoreMesh(core_axis_name="core",
                                  subcore_axis_name="subcore")
    idx2 = idx.reshape(1, n)

    @pl.kernel(out_type=jax.ShapeDtypeStruct((n, _H), emb.dtype), mesh=mesh)
    def k(emb_hbm, i_hbm, o_hbm):
        def body(i_vmem, o_vmem):
            pltpu.sync_copy(emb_hbm.at[i_vmem.at[0]], o_vmem)

        pltpu.emit_pipeline(
            body,
            grid=(n // _GW,),
            in_specs=[pl.BlockSpec((1, _GW), index_map=lambda i: (0, i))],
            out_specs=[pl.BlockSpec((_GW, _H), index_map=lambda i: (i, 0))],
            core_axis_name=("core", "subcore"),
            dimension_semantics=(pltpu.PARALLEL,),
        )(i_hbm, o_hbm)

    return k(emb, idx2)


def kernel(z_e, embedding):
    zf = z_e.reshape(-1, _H)
    eT = embedding.T
    idx, bd = _argmin_call(zf, eT)
    z_q = _sc_gather(embedding, idx.reshape(-1)).reshape(z_e.shape)
    loss = (1.0 + _BETA) * (jnp.sum(bd) / zf.size)
    return z_q, loss
